# baseline (speedup 1.0000x reference)
import jax
import jax.numpy as jnp
from jax.experimental import pallas as pl
from jax.experimental.pallas import tpu as pltpu

_TEMPERATURE = 2.0
_BLOCK_T = 512
_NBUF = 4


def _router_body(x_hbm, wt_ref, b_ref, out_ref, xbuf, sems):
    tokens = x_hbm.shape[0]
    nchunks = tokens // _BLOCK_T

    def copy(i, slot):
        return pltpu.make_async_copy(
            x_hbm.at[pl.ds(i * _BLOCK_T, _BLOCK_T), :], xbuf.at[slot], sems.at[slot]
        )

    for k in range(_NBUF):
        copy(k, k).start()

    def step(i, carry):
        slot = jax.lax.rem(i, _NBUF)
        copy(i, slot).wait()
        logits = jnp.dot(xbuf[slot], wt_ref[...], preferred_element_type=jnp.float32)
        logits = (logits + b_ref[...]) * (1.0 / _TEMPERATURE)
        m = jnp.max(logits, axis=-1, keepdims=True)
        e = jnp.exp(logits - m)
        out_ref[pl.ds(i * _BLOCK_T, _BLOCK_T), :] = e / jnp.sum(e, axis=-1, keepdims=True)

        @pl.when(i + _NBUF < nchunks)
        def _():
            copy(i + _NBUF, slot).start()

        return carry

    jax.lax.fori_loop(0, nchunks, step, 0)


def kernel(x, W, b):
    tokens, d_model = x.shape
    num_experts = W.shape[0]
    wt = W.T
    b2 = b.reshape(1, num_experts)
    return pl.pallas_call(
        _router_body,
        in_specs=[
            pl.BlockSpec(memory_space=pl.ANY),
            pl.BlockSpec((d_model, num_experts), lambda: (0, 0)),
            pl.BlockSpec((1, num_experts), lambda: (0, 0)),
        ],
        out_specs=pl.BlockSpec((tokens, num_experts), lambda: (0, 0)),
        out_shape=jax.ShapeDtypeStruct((tokens, num_experts), jnp.float32),
        scratch_shapes=[
            pltpu.VMEM((_NBUF, _BLOCK_T, d_model), jnp.float32),
            pltpu.SemaphoreType.DMA((_NBUF,)),
        ],
    )(x, wt, b2)
